# baseline (speedup 1.0000x reference)
import functools

import jax
import jax.numpy as jnp
import numpy as np
from jax import lax
from jax.experimental import pallas as pl
from jax.experimental.pallas import tpu as pltpu
from jax.experimental.pallas import tpu_sc as plsc

B, N, D, H, K = 4, 2048, 128, 64, 16
TBL = 384
RB1 = 512
NB1 = N // RB1
RB3 = 512
NB3 = N // RB3
ROWS_B = N * K


def _pre_body(x_ref, xyzq_ref, xyzf_ref,
              phiW_ref, phib_ref, psiW_ref, psib_ref, alW_ref, alb_ref,
              g1W_ref, g1b_ref, p1W_ref, p2W_ref, p2b_ref,
              table_ref, ag_ref, gidx_ref, p2cat_ref):
    b = pl.program_id(0)
    x = x_ref[0]
    xyzq = xyzq_ref[0]
    xyzf = xyzf_ref[0]

    g1W = g1W_ref[...]
    kg = jnp.dot(jnp.dot(x, psiW_ref[...], preferred_element_type=jnp.float32)
                 + psib_ref[...], g1W, preferred_element_type=jnp.float32)
    v = jnp.dot(x, alW_ref[...], preferred_element_type=jnp.float32) + alb_ref[...]
    w = jnp.dot(xyzq, p1W_ref[...], preferred_element_type=jnp.float32)
    table_ref[0] = jnp.concatenate([kg, v, w, jnp.zeros((x.shape[0], TBL - 2 * D - H), jnp.float32)], axis=1)

    q = jnp.dot(x, phiW_ref[...], preferred_element_type=jnp.float32) + phib_ref[...]
    cbias = g1b_ref[...] + jnp.dot(p2b_ref[...], g1W, preferred_element_type=jnp.float32)
    ag_ref[0] = jnp.dot(q, g1W, preferred_element_type=jnp.float32) + cbias
    p2cat_ref[...] = jnp.concatenate(
        [p2W_ref[...],
         jnp.dot(p2W_ref[...], g1W, preferred_element_type=jnp.float32)], axis=1)

    yq = xyzq * xyzq
    yf = xyzf * xyzf
    ssq_q = jnp.sum(yq, axis=1, keepdims=True)
    ssq_f = jnp.transpose(jnp.sum(yf, axis=1, keepdims=True), (1, 0))
    g = lax.dot_general(xyzq, xyzf, (((1,), (1,)), ((), ())),
                        preferred_element_type=jnp.float32)
    d2 = (ssq_q + ssq_f) - 2.0 * g

    iota = lax.broadcasted_iota(jnp.int32, (RB1, N), 1).astype(jnp.float32)
    big = jnp.float32(np.inf)
    nf = jnp.float32(N)
    cols = []
    dm = d2
    for _ in range(K):
        m = jnp.min(dm, axis=1, keepdims=True)
        sel = jnp.where(dm <= m, iota, nf)
        j = jnp.min(sel, axis=1, keepdims=True)
        cols.append(j)
        dm = jnp.where(iota == j, big, dm)
    gidx_ref[0] = jnp.concatenate(cols, axis=1).astype(jnp.int32)


def _stage1(x, xyz, phiW, phib, psiW, psib, alW, alb, g1W, g1b, p1W, p2W, p2b):
    grid = (B, NB1)
    wmap = lambda b, r: (0, 0)
    table, ag, gidx, p2cat = pl.pallas_call(
        _pre_body,
        grid=grid,
        in_specs=[
            pl.BlockSpec((1, RB1, D), lambda b, r: (b, r, 0)),
            pl.BlockSpec((1, RB1, 3), lambda b, r: (b, r, 0)),
            pl.BlockSpec((1, N, 3), lambda b, r: (b, 0, 0)),
            pl.BlockSpec((D, D), wmap),
            pl.BlockSpec((1, D), wmap),
            pl.BlockSpec((D, D), wmap),
            pl.BlockSpec((1, D), wmap),
            pl.BlockSpec((D, D), wmap),
            pl.BlockSpec((1, D), wmap),
            pl.BlockSpec((D, D), wmap),
            pl.BlockSpec((1, D), wmap),
            pl.BlockSpec((3, H), wmap),
            pl.BlockSpec((H, D), wmap),
            pl.BlockSpec((1, D), wmap),
        ],
        out_specs=[
            pl.BlockSpec((1, RB1, TBL), lambda b, r: (b, r, 0)),
            pl.BlockSpec((1, RB1, D), lambda b, r: (b, r, 0)),
            pl.BlockSpec((1, RB1, K), lambda b, r: (b, r, 0)),
            pl.BlockSpec((H, 2 * D), wmap),
        ],
        out_shape=[
            jax.ShapeDtypeStruct((B, N, TBL), jnp.float32),
            jax.ShapeDtypeStruct((B, N, D), jnp.float32),
            jax.ShapeDtypeStruct((B, N, K), jnp.int32),
            jax.ShapeDtypeStruct((H, 2 * D), jnp.float32),
        ],
    )(x, xyz, xyz, phiW, phib, psiW, psib, alW, alb, g1W, g1b, p1W, p2W, p2b)
    return table, ag, gidx, p2cat


_SC_CH = 128


def _make_sc_gather():
    info = plsc.get_sparse_core_info()
    nw = info.num_cores * info.num_subcores
    rpw = ROWS_B // nw
    nch = rpw // _SC_CH
    mesh = plsc.VectorSubcoreMesh(core_axis_name="c", subcore_axis_name="s")

    @functools.partial(
        pl.kernel, mesh=mesh,
        out_type=jax.ShapeDtypeStruct((ROWS_B, TBL), jnp.float32),
        scratch_types=[
            pltpu.VMEM((_SC_CH,), jnp.int32),
            pltpu.VMEM((_SC_CH, TBL), jnp.float32),
            pltpu.SemaphoreType.DMA,
        ],
    )
    def sc_gather(table_hbm, idx_hbm, out_hbm, idx_v, rows_v, sem):
        wid = lax.axis_index("s") * info.num_cores + lax.axis_index("c")
        base = wid * rpw
        for c in range(nch):
            off = base + c * _SC_CH
            pltpu.sync_copy(idx_hbm.at[pl.ds(off, _SC_CH)], idx_v)
            pltpu.async_copy(table_hbm.at[idx_v], rows_v, sem).wait()
            pltpu.sync_copy(rows_v, out_hbm.at[pl.ds(off, _SC_CH)])

    return sc_gather


def _attn_body(ag_ref, wq_ref, g_ref, p2cat_ref, g2W_ref, g2b_ref,
               p1b_ref, p2b_ref, attn_ref, out_ref):
    ag = ag_ref[0]
    wq = wq_ref[0][:, 2 * D:2 * D + H]
    p2cat = p2cat_ref[...]
    g2W = g2W_ref[...]
    g2b = g2b_ref[...]
    p1b = p1b_ref[...]
    p2b = p2b_ref[...]
    scale = jnp.float32(np.sqrt(float(D)))

    rs = [jnp.maximum(wq - g_ref[0, k][:, 2 * D:2 * D + H] + p1b, 0.0)
          for k in range(K)]
    bigr = jnp.concatenate(rs, axis=0)
    bigp = jnp.dot(bigr, p2cat, preferred_element_type=jnp.float32)
    us = [jnp.maximum(ag - g_ref[0, k][:, 0:D]
                      + bigp[k * RB3:(k + 1) * RB3, D:2 * D], 0.0)
          for k in range(K)]
    bigu = jnp.concatenate(us, axis=0)
    biglg = (jnp.dot(bigu, g2W, preferred_element_type=jnp.float32) + g2b) / scale
    logits = [biglg[k * RB3:(k + 1) * RB3] for k in range(K)]
    vpes = [g_ref[0, k][:, D:2 * D] + bigp[k * RB3:(k + 1) * RB3, 0:D] + p2b
            for k in range(K)]

    m = logits[0]
    for k in range(1, K):
        m = jnp.maximum(m, logits[k])
    es = [jnp.exp(lg - m) for lg in logits]
    s = es[0]
    for k in range(1, K):
        s = s + es[k]
    attns = [e / s for e in es]
    attn_ref[0] = jnp.stack(attns, axis=0)
    o = attns[0] * vpes[0]
    for k in range(1, K):
        o = o + attns[k] * vpes[k]
    out_ref[0] = o


def _stage3(ag, wq_tab, g, p2cat, g2W, g2b, p1b, p2b):
    grid = (NB3,)
    wmap = lambda i: (0, 0)
    attn, out = pl.pallas_call(
        _attn_body,
        grid=grid,
        in_specs=[
            pl.BlockSpec((1, RB3, D), lambda i: (i, 0, 0)),
            pl.BlockSpec((1, RB3, TBL), lambda i: (i, 0, 0)),
            pl.BlockSpec((1, K, RB3, TBL), lambda i: (i, 0, 0, 0)),
            pl.BlockSpec((H, 2 * D), wmap),
            pl.BlockSpec((D, D), wmap),
            pl.BlockSpec((1, D), wmap),
            pl.BlockSpec((1, H), wmap),
            pl.BlockSpec((1, D), wmap),
        ],
        out_specs=[
            pl.BlockSpec((1, K, RB3, D), lambda i: (i, 0, 0, 0)),
            pl.BlockSpec((1, RB3, D), lambda i: (i, 0, 0)),
        ],
        out_shape=[
            jax.ShapeDtypeStruct((NB3, K, RB3, D), jnp.float32),
            jax.ShapeDtypeStruct((NB3, RB3, D), jnp.float32),
        ],
    )(ag, wq_tab, g, p2cat, g2W, g2b, p1b, p2b)
    return attn, out


def kernel(input_feature, xyz, phi_W, phi_b, psi_W, psi_b, alpha_W, alpha_b,
           g1_W, g1_b, g2_W, g2_b, p1_W, p1_b, p2_W, p2_b):
    r1 = lambda a: a.reshape(1, -1)
    sc_gather = _make_sc_gather()
    table, ag, gidx, p2cat = _stage1(
        input_feature, xyz, phi_W, r1(phi_b), psi_W, r1(psi_b),
        alpha_W, r1(alpha_b), g1_W, r1(g1_b), p1_W, p2_W, r1(p2_b))
    outs, attns = [], []
    for b in range(B):
        idx_flat = (gidx[b].reshape(NB3, RB3, K)
                    .transpose(0, 2, 1)
                    .reshape(ROWS_B))
        g = sc_gather(table[b], idx_flat)
        attn_b, out_b = _stage3(
            ag[b].reshape(NB3, RB3, D),
            table[b].reshape(NB3, RB3, TBL),
            g.reshape(NB3, K, RB3, TBL),
            p2cat, g2_W, r1(g2_b), r1(p1_b), r1(p2_b))
        attns.append(attn_b.transpose(0, 2, 1, 3))
        outs.append(out_b.reshape(N, D))
    attn = jnp.stack(attns).reshape(B, N, K, D)
    return (jnp.stack(outs), attn)


# speedup vs baseline: 1.2601x; 1.2601x over previous
import functools

import jax
import jax.numpy as jnp
import numpy as np
from jax import lax
from jax.experimental import pallas as pl
from jax.experimental.pallas import tpu as pltpu
from jax.experimental.pallas import tpu_sc as plsc

B, N, D, H, K = 4, 2048, 128, 64, 16
TBL = 384
RB1 = 512
NB1 = N // RB1
RB3 = 512
NB3 = N // RB3
ROWS_B = N * K


def _pre_body(x_ref, xyzq_ref, xyzf_ref,
              phiW_ref, phib_ref, psiW_ref, psib_ref, alW_ref, alb_ref,
              g1W_ref, g1b_ref, p1W_ref, p2W_ref, p2b_ref,
              table_ref, ag_ref, gidx_ref, p2cat_ref):
    b = pl.program_id(0)
    x = x_ref[0]
    xyzq = xyzq_ref[0]
    xyzf = xyzf_ref[0]

    g1W = g1W_ref[...]
    kg = jnp.dot(jnp.dot(x, psiW_ref[...], preferred_element_type=jnp.float32)
                 + psib_ref[...], g1W, preferred_element_type=jnp.float32)
    v = jnp.dot(x, alW_ref[...], preferred_element_type=jnp.float32) + alb_ref[...]
    w = jnp.dot(xyzq, p1W_ref[...], preferred_element_type=jnp.float32)
    table_ref[0] = jnp.concatenate([kg, v, w, jnp.zeros((x.shape[0], TBL - 2 * D - H), jnp.float32)], axis=1)

    q = jnp.dot(x, phiW_ref[...], preferred_element_type=jnp.float32) + phib_ref[...]
    cbias = g1b_ref[...] + jnp.dot(p2b_ref[...], g1W, preferred_element_type=jnp.float32)
    ag_ref[0] = jnp.dot(q, g1W, preferred_element_type=jnp.float32) + cbias
    p2cat_ref[...] = jnp.concatenate(
        [p2W_ref[...],
         jnp.dot(p2W_ref[...], g1W, preferred_element_type=jnp.float32)], axis=1)

    yq = xyzq * xyzq
    yf = xyzf * xyzf
    ssq_q = jnp.sum(yq, axis=1, keepdims=True)
    ssq_f = jnp.transpose(jnp.sum(yf, axis=1, keepdims=True), (1, 0))
    g = lax.dot_general(xyzq, xyzf, (((1,), (1,)), ((), ())),
                        preferred_element_type=jnp.float32)
    d2 = (ssq_q + ssq_f) - 2.0 * g

    iota = lax.broadcasted_iota(jnp.int32, (RB1, N), 1).astype(jnp.float32)
    big = jnp.float32(np.inf)
    nf = jnp.float32(N)
    cols = []
    dm = d2
    for t in range(K):
        m = jnp.min(dm, axis=1, keepdims=True)
        sel = jnp.where(dm <= m, iota, nf)
        j = jnp.min(sel, axis=1, keepdims=True)
        cols.append(j)
        if t < K - 1:
            dm = jnp.where(iota == j, big, dm)
    gidx_ref[0] = jnp.concatenate(cols, axis=1).astype(jnp.int32) + b * N


def _stage1(x, xyz, phiW, phib, psiW, psib, alW, alb, g1W, g1b, p1W, p2W, p2b):
    grid = (1, NB1)
    wmap = lambda b, r: (0, 0)
    table, ag, gidx, p2cat = pl.pallas_call(
        _pre_body,
        grid=grid,
        in_specs=[
            pl.BlockSpec((1, RB1, D), lambda b, r: (b, r, 0)),
            pl.BlockSpec((1, RB1, 3), lambda b, r: (b, r, 0)),
            pl.BlockSpec((1, N, 3), lambda b, r: (b, 0, 0)),
            pl.BlockSpec((D, D), wmap),
            pl.BlockSpec((1, D), wmap),
            pl.BlockSpec((D, D), wmap),
            pl.BlockSpec((1, D), wmap),
            pl.BlockSpec((D, D), wmap),
            pl.BlockSpec((1, D), wmap),
            pl.BlockSpec((D, D), wmap),
            pl.BlockSpec((1, D), wmap),
            pl.BlockSpec((3, H), wmap),
            pl.BlockSpec((H, D), wmap),
            pl.BlockSpec((1, D), wmap),
        ],
        out_specs=[
            pl.BlockSpec((1, RB1, TBL), lambda b, r: (b, r, 0)),
            pl.BlockSpec((1, RB1, D), lambda b, r: (b, r, 0)),
            pl.BlockSpec((1, RB1, K), lambda b, r: (b, r, 0)),
            pl.BlockSpec((H, 2 * D), wmap),
        ],
        out_shape=[
            jax.ShapeDtypeStruct((1, N, TBL), jnp.float32),
            jax.ShapeDtypeStruct((1, N, D), jnp.float32),
            jax.ShapeDtypeStruct((1, N, K), jnp.int32),
            jax.ShapeDtypeStruct((H, 2 * D), jnp.float32),
        ],
    )(x, xyz, xyz, phiW, phib, psiW, psib, alW, alb, g1W, g1b, p1W, p2W, p2b)
    return table, ag, gidx, p2cat


_SC_CH = 128


def _make_sc_gather():
    info = plsc.get_sparse_core_info()
    nw = info.num_cores * info.num_subcores
    rpw = ROWS_B // nw
    nch = rpw // _SC_CH
    mesh = plsc.VectorSubcoreMesh(core_axis_name="c", subcore_axis_name="s")

    @functools.partial(
        pl.kernel, mesh=mesh,
        out_type=jax.ShapeDtypeStruct((ROWS_B, TBL), jnp.float32),
        scratch_types=[
            pltpu.VMEM((_SC_CH,), jnp.int32),
            pltpu.VMEM((_SC_CH, TBL), jnp.float32),
            pltpu.SemaphoreType.DMA,
        ],
    )
    def sc_gather(table_hbm, idx_hbm, out_hbm, idx_v, rows_v, sem):
        wid = lax.axis_index("s") * info.num_cores + lax.axis_index("c")
        base = wid * rpw
        for c in range(nch):
            off = base + c * _SC_CH
            pltpu.sync_copy(idx_hbm.at[pl.ds(off, _SC_CH)], idx_v)
            pltpu.async_copy(table_hbm.at[idx_v], rows_v, sem).wait()
            pltpu.sync_copy(rows_v, out_hbm.at[pl.ds(off, _SC_CH)])

    return sc_gather


def _attn_body(ag_ref, wq_ref, g_ref, p2cat_ref, g2W_ref, g2b_ref,
               p1b_ref, p2b_ref, attn_ref, out_ref):
    ag = ag_ref[0]
    wq = wq_ref[0][:, 2 * D:2 * D + H]
    p2cat = p2cat_ref[...]
    g2W = g2W_ref[...]
    g2b = g2b_ref[...]
    p1b = p1b_ref[...]
    p2b = p2b_ref[...]
    scale = jnp.float32(np.sqrt(float(D)))

    rs = [jnp.maximum(wq - g_ref[0, k][:, 2 * D:2 * D + H] + p1b, 0.0)
          for k in range(K)]
    bigr = jnp.concatenate(rs, axis=0)
    bigp = jnp.dot(bigr, p2cat, preferred_element_type=jnp.float32)
    us = [jnp.maximum(ag - g_ref[0, k][:, 0:D]
                      + bigp[k * RB3:(k + 1) * RB3, D:2 * D], 0.0)
          for k in range(K)]
    bigu = jnp.concatenate(us, axis=0)
    biglg = (jnp.dot(bigu, g2W, preferred_element_type=jnp.float32) + g2b) / scale
    logits = [biglg[k * RB3:(k + 1) * RB3] for k in range(K)]
    vpes = [g_ref[0, k][:, D:2 * D] + bigp[k * RB3:(k + 1) * RB3, 0:D] + p2b
            for k in range(K)]

    m = logits[0]
    for k in range(1, K):
        m = jnp.maximum(m, logits[k])
    es = [jnp.exp(lg - m) for lg in logits]
    s = es[0]
    for k in range(1, K):
        s = s + es[k]
    attns = [e / s for e in es]
    attn_ref[0] = jnp.stack(attns, axis=0)
    o = attns[0] * vpes[0]
    for k in range(1, K):
        o = o + attns[k] * vpes[k]
    out_ref[0] = o


def _stage3(ag, wq_tab, g, p2cat, g2W, g2b, p1b, p2b):
    grid = (NB3,)
    wmap = lambda i: (0, 0)
    attn, out = pl.pallas_call(
        _attn_body,
        grid=grid,
        in_specs=[
            pl.BlockSpec((1, RB3, D), lambda i: (i, 0, 0)),
            pl.BlockSpec((1, RB3, TBL), lambda i: (i, 0, 0)),
            pl.BlockSpec((1, K, RB3, TBL), lambda i: (i, 0, 0, 0)),
            pl.BlockSpec((H, 2 * D), wmap),
            pl.BlockSpec((D, D), wmap),
            pl.BlockSpec((1, D), wmap),
            pl.BlockSpec((1, H), wmap),
            pl.BlockSpec((1, D), wmap),
        ],
        out_specs=[
            pl.BlockSpec((1, K, RB3, D), lambda i: (i, 0, 0, 0)),
            pl.BlockSpec((1, RB3, D), lambda i: (i, 0, 0)),
        ],
        out_shape=[
            jax.ShapeDtypeStruct((NB3, K, RB3, D), jnp.float32),
            jax.ShapeDtypeStruct((NB3, RB3, D), jnp.float32),
        ],
    )(ag, wq_tab, g, p2cat, g2W, g2b, p1b, p2b)
    return attn, out


def kernel(input_feature, xyz, phi_W, phi_b, psi_W, psi_b, alpha_W, alpha_b,
           g1_W, g1_b, g2_W, g2_b, p1_W, p1_b, p2_W, p2_b):
    r1 = lambda a: a.reshape(1, -1)
    sc_gather = _make_sc_gather()
    outs, attns = [], []
    for b in range(B):
        xb = input_feature[b:b + 1]
        zb = xyz[b:b + 1]
        table, ag, gidx, p2cat = _stage1(
            xb, zb, phi_W, r1(phi_b), psi_W, r1(psi_b),
            alpha_W, r1(alpha_b), g1_W, r1(g1_b), p1_W, p2_W, r1(p2_b))
        idx_flat = (gidx.reshape(NB3, RB3, K)
                    .transpose(0, 2, 1)
                    .reshape(ROWS_B))
        g = sc_gather(table.reshape(N, TBL), idx_flat)
        attn_b, out_b = _stage3(
            ag.reshape(NB3, RB3, D),
            table.reshape(NB3, RB3, TBL),
            g.reshape(NB3, K, RB3, TBL),
            p2cat, g2_W, r1(g2_b), r1(p1_b), r1(p2_b))
        attns.append(attn_b.transpose(0, 2, 1, 3))
        outs.append(out_b.reshape(N, D))
    attn = jnp.stack(attns).reshape(B, N, K, D)
    return (jnp.stack(outs), attn)
